# baseline (speedup 1.0000x reference)
import functools

import jax
import jax.numpy as jnp
from jax import lax
from jax.experimental import pallas as pl
from jax.experimental.pallas import tpu as pltpu
from jax.experimental.pallas import tpu_sc as plsc

SUB = 128


@functools.lru_cache(maxsize=None)
def _build(b_total: int, d: int, chunk: int):
  info = plsc.get_sparse_core_info()
  nw = info.num_cores * info.num_subcores
  assert b_total % (nw * chunk) == 0 and chunk % SUB == 0
  b_per_w = b_total // nw
  n_chunk = b_per_w // chunk
  n_sub = chunk // SUB

  mesh = plsc.VectorSubcoreMesh(core_axis_name="c", subcore_axis_name="s")

  @functools.partial(
      pl.kernel,
      out_type=jax.ShapeDtypeStruct((b_total, d), jnp.float32),
      mesh=mesh,
      scratch_types=[
          pltpu.VMEM((chunk,), jnp.int32),
          pltpu.VMEM((chunk, d), jnp.float32),
          pltpu.SemaphoreType.DMA,
      ],
      compiler_params=pltpu.CompilerParams(use_tc_tiling_on_sc=False),
  )
  def gather_kernel(table_hbm, idx_hbm, out_hbm, idx_v, rows_v, sem):
    wid = lax.axis_index("s") * info.num_cores + lax.axis_index("c")
    base = wid * b_per_w

    def body(i, carry):
      b0 = base + i * chunk
      pltpu.sync_copy(idx_hbm.at[pl.ds(b0, chunk)], idx_v)
      copies = []
      for j in range(n_sub):
        copies.append(
            pltpu.async_copy(
                table_hbm.at[idx_v.at[pl.ds(j * SUB, SUB)]],
                rows_v.at[pl.ds(j * SUB, SUB)],
                sem,
            )
        )
      for c in copies:
        c.wait()
      pltpu.sync_copy(rows_v, out_hbm.at[pl.ds(b0, chunk)])
      return carry

    lax.fori_loop(0, n_chunk, body, 0)

  return gather_kernel


def kernel(item_feat_index, emb_table):
  batch, hist = item_feat_index.shape
  _, d = emb_table.shape
  idx = item_feat_index.reshape(-1).astype(jnp.int32)
  out = _build(batch * hist, d, 1024)(emb_table, idx)
  return out.reshape(batch, hist, d)


# speedup vs baseline: 1.0063x; 1.0063x over previous
import functools

import jax
import jax.numpy as jnp
from jax import lax
from jax.experimental import pallas as pl
from jax.experimental.pallas import tpu as pltpu
from jax.experimental.pallas import tpu_sc as plsc

SUB = 128


@functools.lru_cache(maxsize=None)
def _build(b_total: int, d: int, chunk: int):
  info = plsc.get_sparse_core_info()
  nw = info.num_cores * info.num_subcores
  assert b_total % (nw * chunk) == 0 and chunk % SUB == 0
  b_per_w = b_total // nw
  n_chunk = b_per_w // chunk
  assert n_chunk % 2 == 0
  n_half = n_chunk // 2
  n_sub = chunk // SUB

  mesh = plsc.VectorSubcoreMesh(core_axis_name="c", subcore_axis_name="s")

  @functools.partial(
      pl.kernel,
      out_type=jax.ShapeDtypeStruct((b_total, d), jnp.float32),
      mesh=mesh,
      scratch_types=[
          pltpu.VMEM((chunk,), jnp.int32),
          pltpu.VMEM((chunk,), jnp.int32),
          pltpu.VMEM((chunk, d), jnp.float32),
          pltpu.VMEM((chunk, d), jnp.float32),
          pltpu.SemaphoreType.DMA,
          pltpu.SemaphoreType.DMA,
          pltpu.SemaphoreType.DMA,
          pltpu.SemaphoreType.DMA,
      ],
      compiler_params=pltpu.CompilerParams(use_tc_tiling_on_sc=False),
  )
  def gather_kernel(table_hbm, idx_hbm, out_hbm,
                    idx0, idx1, rows0, rows1, sg0, sg1, so0, so1):
    wid = lax.axis_index("s") * info.num_cores + lax.axis_index("c")
    base = wid * b_per_w
    idx_bufs = (idx0, idx1)
    row_bufs = (rows0, rows1)
    sgs = (sg0, sg1)
    sos = (so0, so1)

    def fire_gathers(c, buf):
      pltpu.sync_copy(idx_hbm.at[pl.ds(base + c * chunk, chunk)],
                      idx_bufs[buf])
      for j in range(n_sub):
        pltpu.async_copy(
            table_hbm.at[idx_bufs[buf].at[pl.ds(j * SUB, SUB)]],
            row_bufs[buf].at[pl.ds(j * SUB, SUB)],
            sgs[buf],
        )

    def drain_gathers(buf):
      pltpu.make_async_copy(out_hbm.at[pl.ds(0, chunk)], row_bufs[buf],
                            sgs[buf]).wait()

    def fire_wb(c, buf):
      pltpu.async_copy(row_bufs[buf], out_hbm.at[pl.ds(base + c * chunk, chunk)],
                       sos[buf])

    def drain_wb(buf):
      pltpu.make_async_copy(row_bufs[buf], out_hbm.at[pl.ds(0, chunk)],
                            sos[buf]).wait()

    fire_gathers(0, 0)

    def body(i, carry):
      c0 = 2 * i

      @pl.when(i > 0)
      def _():
        drain_wb(1)

      fire_gathers(c0 + 1, 1)
      drain_gathers(0)
      fire_wb(c0, 0)

      drain_wb(0)

      @pl.when(i < n_half - 1)
      def _():
        fire_gathers(c0 + 2, 0)

      drain_gathers(1)
      fire_wb(c0 + 1, 1)
      return carry

    lax.fori_loop(0, n_half, body, 0)
    drain_wb(1)

  return gather_kernel


def kernel(item_feat_index, emb_table):
  batch, hist = item_feat_index.shape
  _, d = emb_table.shape
  idx = item_feat_index.reshape(-1).astype(jnp.int32)
  out = _build(batch * hist, d, 512)(emb_table, idx)
  return out.reshape(batch, hist, d)


# speedup vs baseline: 1.0064x; 1.0001x over previous
import functools

import jax
import jax.numpy as jnp
from jax import lax
from jax.experimental import pallas as pl
from jax.experimental.pallas import tpu as pltpu
from jax.experimental.pallas import tpu_sc as plsc

SUB = 512


@functools.lru_cache(maxsize=None)
def _build(b_total: int, d: int, chunk: int):
  info = plsc.get_sparse_core_info()
  nw = info.num_cores * info.num_subcores
  assert b_total % (nw * chunk) == 0 and chunk % SUB == 0
  b_per_w = b_total // nw
  n_chunk = b_per_w // chunk
  assert n_chunk % 2 == 0
  n_half = n_chunk // 2
  n_sub = chunk // SUB

  mesh = plsc.VectorSubcoreMesh(core_axis_name="c", subcore_axis_name="s")

  @functools.partial(
      pl.kernel,
      out_type=jax.ShapeDtypeStruct((b_total, d), jnp.float32),
      mesh=mesh,
      scratch_types=[
          pltpu.VMEM((chunk,), jnp.int32),
          pltpu.VMEM((chunk,), jnp.int32),
          pltpu.VMEM((chunk, d), jnp.float32),
          pltpu.VMEM((chunk, d), jnp.float32),
          pltpu.SemaphoreType.DMA,
          pltpu.SemaphoreType.DMA,
          pltpu.SemaphoreType.DMA,
          pltpu.SemaphoreType.DMA,
      ],
      compiler_params=pltpu.CompilerParams(use_tc_tiling_on_sc=False),
  )
  def gather_kernel(table_hbm, idx_hbm, out_hbm,
                    idx0, idx1, rows0, rows1, sg0, sg1, so0, so1):
    wid = lax.axis_index("s") * info.num_cores + lax.axis_index("c")
    base = wid * b_per_w
    idx_bufs = (idx0, idx1)
    row_bufs = (rows0, rows1)
    sgs = (sg0, sg1)
    sos = (so0, so1)

    def fire_gathers(c, buf):
      pltpu.sync_copy(idx_hbm.at[pl.ds(base + c * chunk, chunk)],
                      idx_bufs[buf])
      for j in range(n_sub):
        pltpu.async_copy(
            table_hbm.at[idx_bufs[buf].at[pl.ds(j * SUB, SUB)]],
            row_bufs[buf].at[pl.ds(j * SUB, SUB)],
            sgs[buf],
        )

    def drain_gathers(buf):
      pltpu.make_async_copy(out_hbm.at[pl.ds(0, chunk)], row_bufs[buf],
                            sgs[buf]).wait()

    def fire_wb(c, buf):
      pltpu.async_copy(row_bufs[buf], out_hbm.at[pl.ds(base + c * chunk, chunk)],
                       sos[buf])

    def drain_wb(buf):
      pltpu.make_async_copy(row_bufs[buf], out_hbm.at[pl.ds(0, chunk)],
                            sos[buf]).wait()

    fire_gathers(0, 0)

    def body(i, carry):
      c0 = 2 * i

      @pl.when(i > 0)
      def _():
        drain_wb(1)

      fire_gathers(c0 + 1, 1)
      drain_gathers(0)
      fire_wb(c0, 0)

      drain_wb(0)

      @pl.when(i < n_half - 1)
      def _():
        fire_gathers(c0 + 2, 0)

      drain_gathers(1)
      fire_wb(c0 + 1, 1)
      return carry

    lax.fori_loop(0, n_half, body, 0)
    drain_wb(1)

  return gather_kernel


def kernel(item_feat_index, emb_table):
  batch, hist = item_feat_index.shape
  _, d = emb_table.shape
  idx = item_feat_index.reshape(-1).astype(jnp.int32)
  out = _build(batch * hist, d, 512)(emb_table, idx)
  return out.reshape(batch, hist, d)


# speedup vs baseline: 1.0093x; 1.0029x over previous
import functools

import jax
import jax.numpy as jnp
from jax import lax
from jax.experimental import pallas as pl
from jax.experimental.pallas import tpu as pltpu
from jax.experimental.pallas import tpu_sc as plsc


@functools.lru_cache(maxsize=None)
def _build(batch: int, hist: int, d: int, cr: int):
  info = plsc.get_sparse_core_info()
  nw = info.num_cores * info.num_subcores
  assert batch % (nw * cr) == 0
  rows_per_w = batch // nw
  n_chunk = rows_per_w // cr
  assert n_chunk % 2 == 0
  n_half = n_chunk // 2

  mesh = plsc.VectorSubcoreMesh(core_axis_name="c", subcore_axis_name="s")

  @functools.partial(
      pl.kernel,
      out_type=jax.ShapeDtypeStruct((batch, hist, d), jnp.float32),
      mesh=mesh,
      scratch_types=[
          pltpu.VMEM((cr, hist), jnp.int32),
          pltpu.VMEM((cr, hist), jnp.int32),
          pltpu.VMEM((cr, hist, d), jnp.float32),
          pltpu.VMEM((cr, hist, d), jnp.float32),
          pltpu.SemaphoreType.DMA,
          pltpu.SemaphoreType.DMA,
          pltpu.SemaphoreType.DMA,
          pltpu.SemaphoreType.DMA,
      ],
      compiler_params=pltpu.CompilerParams(use_tc_tiling_on_sc=False),
  )
  def gather_kernel(table_hbm, idx_hbm, out_hbm,
                    idx0, idx1, rows0, rows1, sg0, sg1, so0, so1):
    wid = lax.axis_index("s") * info.num_cores + lax.axis_index("c")
    base = wid * rows_per_w
    idx_bufs = (idx0, idx1)
    row_bufs = (rows0, rows1)
    sgs = (sg0, sg1)
    sos = (so0, so1)

    def fire_gathers(c, buf):
      pltpu.sync_copy(idx_hbm.at[pl.ds(base + c * cr, cr)], idx_bufs[buf])
      for r in range(cr):
        pltpu.async_copy(
            table_hbm.at[idx_bufs[buf].at[r]],
            row_bufs[buf].at[r],
            sgs[buf],
        )

    def drain_gathers(buf):
      pltpu.make_async_copy(out_hbm.at[pl.ds(0, cr)], row_bufs[buf],
                            sgs[buf]).wait()

    def fire_wb(c, buf):
      pltpu.async_copy(row_bufs[buf], out_hbm.at[pl.ds(base + c * cr, cr)],
                       sos[buf])

    def drain_wb(buf):
      pltpu.make_async_copy(row_bufs[buf], out_hbm.at[pl.ds(0, cr)],
                            sos[buf]).wait()

    fire_gathers(0, 0)

    def body(i, carry):
      c0 = 2 * i

      @pl.when(i > 0)
      def _():
        drain_wb(1)

      fire_gathers(c0 + 1, 1)
      drain_gathers(0)
      fire_wb(c0, 0)

      drain_wb(0)

      @pl.when(i < n_half - 1)
      def _():
        fire_gathers(c0 + 2, 0)

      drain_gathers(1)
      fire_wb(c0 + 1, 1)
      return carry

    lax.fori_loop(0, n_half, body, 0)
    drain_wb(1)

  return gather_kernel


def kernel(item_feat_index, emb_table):
  batch, hist = item_feat_index.shape
  _, d = emb_table.shape
  idx = item_feat_index
  if idx.dtype != jnp.int32:
    idx = idx.astype(jnp.int32)
  return _build(batch, hist, d, 4)(emb_table, idx)


# speedup vs baseline: 1.3381x; 1.3258x over previous
import functools

import jax
import jax.numpy as jnp
from jax import lax
from jax.experimental import pallas as pl
from jax.experimental.pallas import tpu as pltpu
from jax.experimental.pallas import tpu_sc as plsc

LANE = 128
PAD = 128


@functools.lru_cache(maxsize=None)
def _build(n_rows: int, d: int, cr: int):
  info = plsc.get_sparse_core_info()
  nw = info.num_cores * info.num_subcores
  assert n_rows % (nw * cr) == 0
  rows_per_w = n_rows // nw
  n_chunk = rows_per_w // cr
  assert n_chunk % 2 == 0
  n_half = n_chunk // 2
  chunk = cr * LANE

  mesh = plsc.VectorSubcoreMesh(core_axis_name="c", subcore_axis_name="s")

  @functools.partial(
      pl.kernel,
      out_type=jax.ShapeDtypeStruct((n_rows * LANE, PAD), jnp.float32),
      mesh=mesh,
      scratch_types=[
          pltpu.VMEM((cr, LANE), jnp.int32),
          pltpu.VMEM((cr, LANE), jnp.int32),
          pltpu.VMEM((chunk, d), jnp.float32),
          pltpu.VMEM((chunk, d), jnp.float32),
          pltpu.SemaphoreType.DMA,
          pltpu.SemaphoreType.DMA,
          pltpu.SemaphoreType.DMA,
          pltpu.SemaphoreType.DMA,
      ],
      compiler_params=pltpu.CompilerParams(use_tc_tiling_on_sc=False),
  )
  def gather_kernel(table_hbm, idx_hbm, out_hbm,
                    idx0, idx1, rows0, rows1, sg0, sg1, so0, so1):
    wid = lax.axis_index("s") * info.num_cores + lax.axis_index("c")
    row_base = wid * rows_per_w
    flat_base = row_base * LANE
    idx_bufs = (idx0, idx1)
    row_bufs = (rows0, rows1)
    sgs = (sg0, sg1)
    sos = (so0, so1)

    def fire_gathers(c, buf):
      pltpu.sync_copy(idx_hbm.at[pl.ds(row_base + c * cr, cr)],
                      idx_bufs[buf])
      for r in range(cr):
        pltpu.async_copy(
            table_hbm.at[idx_bufs[buf].at[r]],
            row_bufs[buf].at[pl.ds(r * LANE, LANE)],
            sgs[buf],
        )

    def drain_gathers(buf):
      pltpu.make_async_copy(out_hbm.at[pl.ds(0, chunk), pl.ds(0, d)],
                            row_bufs[buf], sgs[buf]).wait()

    def fire_wb(c, buf):
      pltpu.async_copy(
          row_bufs[buf],
          out_hbm.at[pl.ds(flat_base + c * chunk, chunk), pl.ds(0, d)],
          sos[buf])

    def drain_wb(buf):
      pltpu.make_async_copy(row_bufs[buf],
                            out_hbm.at[pl.ds(0, chunk), pl.ds(0, d)],
                            sos[buf]).wait()

    fire_gathers(0, 0)

    def body(i, carry):
      c0 = 2 * i

      @pl.when(i > 0)
      def _():
        drain_wb(1)

      fire_gathers(c0 + 1, 1)
      drain_gathers(0)
      fire_wb(c0, 0)

      drain_wb(0)

      @pl.when(i < n_half - 1)
      def _():
        fire_gathers(c0 + 2, 0)

      drain_gathers(1)
      fire_wb(c0 + 1, 1)
      return carry

    lax.fori_loop(0, n_half, body, 0)
    drain_wb(1)

  return gather_kernel


def kernel(item_feat_index, emb_table):
  batch, hist = item_feat_index.shape
  _, d = emb_table.shape
  idx = item_feat_index
  if idx.dtype != jnp.int32:
    idx = idx.astype(jnp.int32)
  n_flat = batch * hist
  idx2 = idx.reshape(n_flat // LANE, LANE)
  padded = _build(n_flat // LANE, d, 5)(emb_table, idx2)
  return padded.reshape(batch, hist, PAD)[:, :, :d]
